# Initial kernel scaffold; baseline (speedup 1.0000x reference)
import jax
import jax.numpy as jnp
from jax.experimental import pallas as pl


def kernel(logits):
    raise NotImplementedError("write your pallas kernel here")


# speedup vs baseline: 1.2676x; 1.2676x over previous
import functools

import jax
import jax.numpy as jnp
from jax import lax
from jax.experimental import pallas as pl
from jax.experimental.pallas import tpu as pltpu
from jax.experimental.pallas import tpu_sc as plsc

E = 64
K = 8
NT = 32768
L = 16
NC, NS = 2, 16
NW = NC * NS
TSUB = NT // NW


def _router_body(logits_hbm, w_hbm, ids_hbm, hist_hbm, lg_v, w_v, ids_v, hist_v):
    wid = lax.axis_index("s") * NC + lax.axis_index("c")
    base = wid * TSUB

    pltpu.sync_copy(logits_hbm.at[pl.ds(base * E, TSUB * E)], lg_v)

    lane = lax.iota(jnp.int32, L)
    low8 = lane < 8
    ones = jnp.ones((L,), jnp.float32)
    zeros = jnp.zeros((L,), jnp.float32)
    group_ids = [lane + g * L for g in range(E // L)]

    for g in range(E // L):
        hist_v[pl.ds(g * L, L)] = zeros

    def merge(a, b):
        av, ai = a
        bv, bi = b
        mv = jnp.where(low8, av, lax.rev(bv, (0,)))
        mi = jnp.where(low8, ai, lax.rev(bi, (0,)))
        return plsc.sort_key_val(mv, mi, descending=True)

    def token_body(t, _):
        o = t * E
        runs = []
        for g in range(E // L):
            v = lg_v[pl.ds(o + g * L, L)]
            runs.append(plsc.sort_key_val(v, group_ids[g], descending=True))
        fv, fi = merge(merge(runs[0], runs[1]), merge(runs[2], runs[3]))
        w = jnp.exp(fv - jnp.max(fv))
        w = jnp.where(low8, w, 0.0)
        w = w / jnp.sum(w)
        out_idx = t * K + lane
        plsc.store_scatter(w_v, [out_idx], w, mask=low8)
        plsc.store_scatter(ids_v, [out_idx], fi, mask=low8)
        plsc.addupdate_scatter(hist_v, [fi], ones, mask=low8)
        return _

    lax.fori_loop(0, TSUB, token_body, None)

    pltpu.sync_copy(w_v, w_hbm.at[pl.ds(base * K, TSUB * K)])
    pltpu.sync_copy(ids_v, ids_hbm.at[pl.ds(base * K, TSUB * K)])
    pltpu.sync_copy(hist_v, hist_hbm.at[pl.ds(wid * E, E)])


_router = functools.partial(
    pl.kernel,
    mesh=plsc.VectorSubcoreMesh(
        core_axis_name="c", subcore_axis_name="s", num_cores=NC, num_subcores=NS
    ),
    out_type=(
        jax.ShapeDtypeStruct((NT * K,), jnp.float32),
        jax.ShapeDtypeStruct((NT * K,), jnp.int32),
        jax.ShapeDtypeStruct((NW * E,), jnp.float32),
    ),
    scratch_types=[
        pltpu.VMEM((TSUB * E,), jnp.float32),
        pltpu.VMEM((TSUB * K,), jnp.float32),
        pltpu.VMEM((TSUB * K,), jnp.int32),
        pltpu.VMEM((E,), jnp.float32),
    ],
    compiler_params=pltpu.CompilerParams(needs_layout_passes=False),
)(_router_body)


def _hist_reduce_body(p_ref, o_ref):
    o_ref[...] = jnp.sum(p_ref[...], axis=0)


def kernel(logits):
    w_flat, ids_flat, partials = _router(logits.reshape(-1))
    tokens_per_expert = pl.pallas_call(
        _hist_reduce_body,
        out_shape=jax.ShapeDtypeStruct((E,), jnp.float32),
    )(partials.reshape(NW, E))
    return (
        logits,
        w_flat.reshape(NT, K),
        ids_flat.reshape(NT, K),
        tokens_per_expert,
    )
